# Initial kernel scaffold; baseline (speedup 1.0000x reference)
import functools

import jax
import jax.numpy as jnp
from jax import lax
from jax.experimental import pallas as pl
from jax.experimental.pallas import tpu as pltpu
from jax.experimental.pallas import tpu_sc as plsc

N = 10000
E = 320000
D = 128
R = 8

NC = 2
NS = 16
NW = NC * NS
CHUNK = 128
NCHUNK = 80
EPT = NCHUNK * CHUNK
EPAD = NW * EPT
TROWS = 632
NACC = NS * TROWS


def _sc_scatter_body(xt_hbm, src_hbm, rel_hbm, dst_hbm, zeros_hbm, out_hbm,
                     src_v, rel_v, gidx_v, dst_v, rows_v, acc_sh, sem):
    c = lax.axis_index("c")
    s = lax.axis_index("s")
    wid = s * NC + c

    pltpu.sync_copy(src_hbm.at[wid], src_v)
    pltpu.sync_copy(rel_hbm.at[wid], rel_v)
    pltpu.sync_copy(dst_hbm.at[wid], dst_v)

    pltpu.sync_copy(zeros_hbm, acc_sh.at[pl.ds(s * TROWS, TROWS)])

    def _gidx_chunk(j, carry):
        for k in range(CHUNK // 16):
            sl = pl.ds(k * 16, 16)
            gidx_v[j, sl] = rel_v[j, sl] * N + src_v[j, sl]
        return carry

    lax.fori_loop(0, NCHUNK, _gidx_chunk, 0)

    plsc.subcore_barrier()

    def _step(j, carry):
        pltpu.async_copy(xt_hbm.at[gidx_v.at[j]], rows_v, sem).wait()
        pltpu.sync_copy(rows_v, acc_sh.at[dst_v.at[j]], add=True)
        return carry

    lax.fori_loop(0, NCHUNK, _step, 0)

    plsc.subcore_barrier()

    off = c * NACC + s * TROWS
    pltpu.sync_copy(acc_sh.at[pl.ds(s * TROWS, TROWS)],
                    out_hbm.at[pl.ds(off, TROWS)])


_sc_scatter = pl.kernel(
    _sc_scatter_body,
    out_type=jax.ShapeDtypeStruct((NC * NACC, D), jnp.float32),
    mesh=plsc.VectorSubcoreMesh(core_axis_name="c", subcore_axis_name="s"),
    scratch_types=[
        pltpu.VMEM((NCHUNK, CHUNK), jnp.int32),
        pltpu.VMEM((NCHUNK, CHUNK), jnp.int32),
        pltpu.VMEM((NCHUNK, CHUNK), jnp.int32),
        pltpu.VMEM((NCHUNK, CHUNK), jnp.int32),
        pltpu.VMEM((CHUNK, D), jnp.float32),
        pltpu.VMEM_SHARED((NACC, D), jnp.float32),
        pltpu.SemaphoreType.DMA,
    ],
)


def _rxw_body(h_ref, w_ref, o_ref):
    o_ref[0] = jnp.dot(h_ref[...], w_ref[0],
                       preferred_element_type=jnp.float32)


def _rxw(h, w):
    return pl.pallas_call(
        _rxw_body,
        grid=(R,),
        in_specs=[
            pl.BlockSpec((N, D), lambda r: (0, 0)),
            pl.BlockSpec((1, D, D), lambda r: (r, 0, 0)),
        ],
        out_specs=pl.BlockSpec((1, N, D), lambda r: (r, 0, 0)),
        out_shape=jax.ShapeDtypeStruct((R, N, D), jnp.float32),
    )(h, w)


def _add_relu_body(p0_ref, p1_ref, o_ref):
    o_ref[...] = jnp.maximum(p0_ref[...] + p1_ref[...], 0.0)


def _add_relu(p0, p1):
    return pl.pallas_call(
        _add_relu_body,
        out_shape=jax.ShapeDtypeStruct((N, D), jnp.float32),
    )(p0, p1)


def _final_body(p0_ref, p1_ref, aw1_ref, ab1_ref, aw2_ref, ab2_ref,
                h_ref, att_ref):
    h2 = p0_ref[...] + p1_ref[...]
    h_ref[...] = h2
    m = jnp.mean(h2, axis=0, keepdims=True)
    a = jnp.maximum(
        jnp.dot(m, aw1_ref[...], preferred_element_type=jnp.float32)
        + ab1_ref[...], 0.0)
    att_ref[...] = jax.nn.sigmoid(
        jnp.dot(a, aw2_ref[...], preferred_element_type=jnp.float32)
        + ab2_ref[...])


def _final(p0, p1, aw1, ab1, aw2, ab2):
    return pl.pallas_call(
        _final_body,
        out_shape=(
            jax.ShapeDtypeStruct((N, D), jnp.float32),
            jax.ShapeDtypeStruct((1, 10), jnp.float32),
        ),
    )(p0, p1, aw1, ab1.reshape(1, -1), aw2, ab2.reshape(1, -1))


def kernel(x, edge_index, edge_type, w1, w2, aw1, ab1, aw2, ab2):
    src = edge_index[0]
    dst = edge_index[1]

    pad = EPAD - E
    srcb = jnp.concatenate([src, jnp.zeros((pad,), jnp.int32)]) \
        .reshape(NW, NCHUNK, CHUNK)
    relb = jnp.concatenate([edge_type, jnp.zeros((pad,), jnp.int32)]) \
        .reshape(NW, NCHUNK, CHUNK)
    dstb = jnp.concatenate([dst, jnp.full((pad,), N, jnp.int32)]) \
        .reshape(NW, NCHUNK, CHUNK)
    ztile = jnp.zeros((TROWS, D), jnp.float32)

    def layer(h, w):
        xt = _rxw(h, w).reshape(R * N, D)
        p = _sc_scatter(xt, srcb, relb, dstb, ztile)
        return p[:N], p[NACC:NACC + N]

    p0, p1 = layer(x, w1)
    h1 = _add_relu(p0, p1)
    q0, q1 = layer(h1, w2)
    h2, att = _final(q0, q1, aw1, ab1, aw2, ab2)
    return (h2, att)


# speedup vs baseline: 9.7996x; 9.7996x over previous
import functools

import jax
import jax.numpy as jnp
from jax import lax
from jax.experimental import pallas as pl
from jax.experimental.pallas import tpu as pltpu
from jax.experimental.pallas import tpu_sc as plsc

N = 10000
E = 320000
D = 128
R = 8

NC = 2
NS = 16
NW = NC * NS
CHUNK = 128
NCHUNK = 80
EPT = NCHUNK * CHUNK
EPAD = NW * EPT
TROWS = 640
NACC = NS * TROWS


def _sc_scatter_body(xt_hbm, src_hbm, rel_hbm, dst_hbm, zeros_hbm, out_hbm,
                     gidx_v, dst_v, rows_v, acc_sh, sem):
    c = lax.axis_index("c")
    s = lax.axis_index("s")
    wid = s * NC + c

    pltpu.sync_copy(src_hbm.at[wid], gidx_v)
    pltpu.sync_copy(rel_hbm.at[wid], dst_v)

    pltpu.sync_copy(zeros_hbm, acc_sh.at[pl.ds(s * TROWS, TROWS)])

    def _gidx_chunk(j, carry):
        for k in range(CHUNK // 16):
            sl = pl.ds(k * 16, 16)
            gidx_v[j, sl] = dst_v[j, sl] * N + gidx_v[j, sl]
        return carry

    lax.fori_loop(0, NCHUNK, _gidx_chunk, 0)

    pltpu.sync_copy(dst_hbm.at[wid], dst_v)

    plsc.subcore_barrier()

    def _step(j, carry):
        pltpu.async_copy(xt_hbm.at[gidx_v.at[j]], rows_v, sem).wait()
        pltpu.sync_copy(rows_v, acc_sh.at[dst_v.at[j]], add=True)
        return carry

    lax.fori_loop(0, NCHUNK, _step, 0)

    plsc.subcore_barrier()

    off = c * NACC + s * TROWS
    pltpu.sync_copy(acc_sh.at[pl.ds(s * TROWS, TROWS)],
                    out_hbm.at[pl.ds(off, TROWS)])


_sc_scatter = pl.kernel(
    _sc_scatter_body,
    out_type=jax.ShapeDtypeStruct((NC * NACC, D), jnp.float32),
    mesh=plsc.VectorSubcoreMesh(core_axis_name="c", subcore_axis_name="s"),
    scratch_types=[
        pltpu.VMEM((NCHUNK, CHUNK), jnp.int32),
        pltpu.VMEM((NCHUNK, CHUNK), jnp.int32),
        pltpu.VMEM((CHUNK, D), jnp.float32),
        pltpu.VMEM_SHARED((NACC, D), jnp.float32),
        pltpu.SemaphoreType.DMA,
    ],
)


def _rxw_body(h_ref, w_ref, o_ref):
    o_ref[0] = jnp.dot(h_ref[...], w_ref[0],
                       preferred_element_type=jnp.float32)


def _rxw(h, w):
    return pl.pallas_call(
        _rxw_body,
        grid=(R,),
        in_specs=[
            pl.BlockSpec((N, D), lambda r: (0, 0)),
            pl.BlockSpec((1, D, D), lambda r: (r, 0, 0)),
        ],
        out_specs=pl.BlockSpec((1, N, D), lambda r: (r, 0, 0)),
        out_shape=jax.ShapeDtypeStruct((R, N, D), jnp.float32),
    )(h, w)


def _add_relu_body(p0_ref, p1_ref, o_ref):
    o_ref[...] = jnp.maximum(p0_ref[...] + p1_ref[...], 0.0)


def _add_relu(p0, p1):
    return pl.pallas_call(
        _add_relu_body,
        out_shape=jax.ShapeDtypeStruct((N, D), jnp.float32),
    )(p0, p1)


def _final_body(p0_ref, p1_ref, aw1_ref, ab1_ref, aw2_ref, ab2_ref,
                h_ref, att_ref):
    h2 = p0_ref[...] + p1_ref[...]
    h_ref[...] = h2
    m = jnp.mean(h2, axis=0, keepdims=True)
    a = jnp.maximum(
        jnp.dot(m, aw1_ref[...], preferred_element_type=jnp.float32)
        + ab1_ref[...], 0.0)
    att_ref[...] = jax.nn.sigmoid(
        jnp.dot(a, aw2_ref[...], preferred_element_type=jnp.float32)
        + ab2_ref[...])


def _final(p0, p1, aw1, ab1, aw2, ab2):
    return pl.pallas_call(
        _final_body,
        out_shape=(
            jax.ShapeDtypeStruct((N, D), jnp.float32),
            jax.ShapeDtypeStruct((1, 10), jnp.float32),
        ),
    )(p0, p1, aw1, ab1.reshape(1, -1), aw2, ab2.reshape(1, -1))


def kernel(x, edge_index, edge_type, w1, w2, aw1, ab1, aw2, ab2):
    src = edge_index[0]
    dst = edge_index[1]

    pad = EPAD - E
    srcb = jnp.concatenate([src, jnp.zeros((pad,), jnp.int32)]) \
        .reshape(NW, NCHUNK, CHUNK)
    relb = jnp.concatenate([edge_type, jnp.zeros((pad,), jnp.int32)]) \
        .reshape(NW, NCHUNK, CHUNK)
    dstb = jnp.concatenate([dst, jnp.full((pad,), N, jnp.int32)]) \
        .reshape(NW, NCHUNK, CHUNK)
    ztile = jnp.zeros((TROWS, D), jnp.float32)

    def layer(h, w):
        xt = _rxw(h, w).reshape(R * N, D)
        p = _sc_scatter(xt, srcb, relb, dstb, ztile)
        return p[:N], p[NACC:NACC + N]

    p0, p1 = layer(x, w1)
    h1 = _add_relu(p0, p1)
    q0, q1 = layer(h1, w2)
    h2, att = _final(q0, q1, aw1, ab1, aw2, ab2)
    return (h2, att)


# speedup vs baseline: 10.3515x; 1.0563x over previous
import functools

import jax
import jax.numpy as jnp
from jax import lax
from jax.experimental import pallas as pl
from jax.experimental.pallas import tpu as pltpu
from jax.experimental.pallas import tpu_sc as plsc

N = 10000
E = 320000
D = 128
R = 8

NC = 2
NS = 16
NW = NC * NS
CHUNK = 64
NCHUNK = 160
SB = 4
SBC = NCHUNK // SB
EPT = NCHUNK * CHUNK
EPAD = NW * EPT
TROWS = 640
NACC = NS * TROWS


def _sc_scatter_body(xt_hbm, src_hbm, rel_hbm, dst_hbm, zeros_hbm, out_hbm,
                     gidx_v, dst_v, rows0, rows1, acc_sh, sem0, sem1):
    c = lax.axis_index("c")
    s = lax.axis_index("s")
    wid = s * NC + c

    pltpu.sync_copy(zeros_hbm, acc_sh.at[pl.ds(s * TROWS, TROWS)])
    plsc.subcore_barrier()

    dummy = xt_hbm.at[pl.ds(0, CHUNK)]

    for b in range(SB):
        pltpu.sync_copy(src_hbm.at[wid, b], gidx_v)
        pltpu.sync_copy(rel_hbm.at[wid, b], dst_v)

        def _gidx_chunk(j, carry):
            for k in range(CHUNK // 16):
                sl = pl.ds(k * 16, 16)
                gidx_v[j, sl] = dst_v[j, sl] * N + gidx_v[j, sl]
            return carry

        lax.fori_loop(0, SBC, _gidx_chunk, 0)
        pltpu.sync_copy(dst_hbm.at[wid, b], dst_v)

        pltpu.async_copy(xt_hbm.at[gidx_v.at[0]], rows0, sem0)
        pltpu.async_copy(xt_hbm.at[gidx_v.at[1]], rows1, sem1)

        def _step(i, carry):
            j = 2 * i
            pltpu.make_async_copy(dummy, rows0, sem0).wait()
            pltpu.sync_copy(rows0, acc_sh.at[dst_v.at[j]], add=True)
            pltpu.async_copy(xt_hbm.at[gidx_v.at[j + 2]], rows0, sem0)
            pltpu.make_async_copy(dummy, rows1, sem1).wait()
            pltpu.sync_copy(rows1, acc_sh.at[dst_v.at[j + 1]], add=True)
            pltpu.async_copy(xt_hbm.at[gidx_v.at[j + 3]], rows1, sem1)
            return carry

        lax.fori_loop(0, SBC // 2 - 1, _step, 0)

        pltpu.make_async_copy(dummy, rows0, sem0).wait()
        pltpu.sync_copy(rows0, acc_sh.at[dst_v.at[SBC - 2]], add=True)
        pltpu.make_async_copy(dummy, rows1, sem1).wait()
        pltpu.sync_copy(rows1, acc_sh.at[dst_v.at[SBC - 1]], add=True)

    plsc.subcore_barrier()

    off = c * NACC + s * TROWS
    pltpu.sync_copy(acc_sh.at[pl.ds(s * TROWS, TROWS)],
                    out_hbm.at[pl.ds(off, TROWS)])


_sc_scatter = pl.kernel(
    _sc_scatter_body,
    out_type=jax.ShapeDtypeStruct((NC * NACC, D), jnp.float32),
    mesh=plsc.VectorSubcoreMesh(core_axis_name="c", subcore_axis_name="s"),
    scratch_types=[
        pltpu.VMEM((SBC, CHUNK), jnp.int32),
        pltpu.VMEM((SBC, CHUNK), jnp.int32),
        pltpu.VMEM((CHUNK, D), jnp.float32),
        pltpu.VMEM((CHUNK, D), jnp.float32),
        pltpu.VMEM_SHARED((NACC, D), jnp.float32),
        pltpu.SemaphoreType.DMA,
        pltpu.SemaphoreType.DMA,
    ],
)


def _rxw_body(h_ref, w_ref, o_ref):
    o_ref[0] = jnp.dot(h_ref[...], w_ref[0],
                       preferred_element_type=jnp.float32)


def _rxw(h, w):
    return pl.pallas_call(
        _rxw_body,
        grid=(R,),
        in_specs=[
            pl.BlockSpec((N, D), lambda r: (0, 0)),
            pl.BlockSpec((1, D, D), lambda r: (r, 0, 0)),
        ],
        out_specs=pl.BlockSpec((1, N, D), lambda r: (r, 0, 0)),
        out_shape=jax.ShapeDtypeStruct((R, N, D), jnp.float32),
    )(h, w)


def _add_relu_body(p0_ref, p1_ref, o_ref):
    o_ref[...] = jnp.maximum(p0_ref[...] + p1_ref[...], 0.0)


def _add_relu(p0, p1):
    return pl.pallas_call(
        _add_relu_body,
        out_shape=jax.ShapeDtypeStruct((N, D), jnp.float32),
    )(p0, p1)


def _final_body(p0_ref, p1_ref, aw1_ref, ab1_ref, aw2_ref, ab2_ref,
                h_ref, att_ref):
    h2 = p0_ref[...] + p1_ref[...]
    h_ref[...] = h2
    m = jnp.mean(h2, axis=0, keepdims=True)
    a = jnp.maximum(
        jnp.dot(m, aw1_ref[...], preferred_element_type=jnp.float32)
        + ab1_ref[...], 0.0)
    att_ref[...] = jax.nn.sigmoid(
        jnp.dot(a, aw2_ref[...], preferred_element_type=jnp.float32)
        + ab2_ref[...])


def _final(p0, p1, aw1, ab1, aw2, ab2):
    return pl.pallas_call(
        _final_body,
        out_shape=(
            jax.ShapeDtypeStruct((N, D), jnp.float32),
            jax.ShapeDtypeStruct((1, 10), jnp.float32),
        ),
    )(p0, p1, aw1, ab1.reshape(1, -1), aw2, ab2.reshape(1, -1))


def kernel(x, edge_index, edge_type, w1, w2, aw1, ab1, aw2, ab2):
    src = edge_index[0]
    dst = edge_index[1]

    pad = EPAD - E
    srcb = jnp.concatenate([src, jnp.zeros((pad,), jnp.int32)]) \
        .reshape(NW, SB, SBC, CHUNK)
    relb = jnp.concatenate([edge_type, jnp.zeros((pad,), jnp.int32)]) \
        .reshape(NW, SB, SBC, CHUNK)
    dstb = jnp.concatenate([dst, jnp.full((pad,), N, jnp.int32)]) \
        .reshape(NW, SB, SBC, CHUNK)
    ztile = jnp.zeros((TROWS, D), jnp.float32)

    def layer(h, w):
        xt = _rxw(h, w).reshape(R * N, D)
        p = _sc_scatter(xt, srcb, relb, dstb, ztile)
        return p[:N], p[NACC:NACC + N]

    p0, p1 = layer(x, w1)
    h1 = _add_relu(p0, p1)
    q0, q1 = layer(h1, w2)
    h2, att = _final(q0, q1, aw1, ab1, aw2, ab2)
    return (h2, att)
